# baseline (speedup 1.0000x reference)
import jax
import jax.numpy as jnp
from jax.experimental import pallas as pl
from jax.experimental.pallas import tpu as pltpu

_VOCAB = 8192
_DICT = 64
_OUT = 384
_TOK_BLK = 64


def _vq_kernel(x_ref, w_ref, d_ref, ids_ref, score_ref, logit_ref,
               quant_ref, loss_ref):
    x = x_ref[...]
    w = w_ref[...]
    d = d_ref[...]
    cont = jax.lax.dot_general(
        x, w, (((1,), (1,)), ((), ())),
        preferred_element_type=jnp.float32)
    logit = jax.lax.dot_general(
        cont, d, (((1,), (1,)), ((), ())),
        preferred_element_type=jnp.float32)
    logit_ref[...] = logit

    m = jnp.max(logit, axis=1, keepdims=True)
    e = jnp.exp(logit - m)
    score_ref[...] = e / jnp.sum(e, axis=1, keepdims=True)

    iota = jax.lax.broadcasted_iota(jnp.int32, logit.shape, 1)
    ids = jnp.min(jnp.where(logit == m, iota, _VOCAB), axis=1)
    ids_ref[0, 0, :] = ids

    onehot = (iota == ids[:, None]).astype(jnp.float32)
    quant = jax.lax.dot_general(
        onehot, d, (((1,), (0,)), ((), ())),
        preferred_element_type=jnp.float32)
    quant_ref[...] = quant

    diff = cont - quant
    part = jnp.sum(diff * diff).reshape(1, 1)

    @pl.when(pl.program_id(0) == 0)
    def _():
        loss_ref[...] = jnp.zeros((1, 1), jnp.float32)

    loss_ref[...] += part


def kernel(x, W_out, dictionary):
    B, S, _ = x.shape
    n_tok = B * S
    nb = n_tok // _TOK_BLK
    x2d = x.reshape(n_tok, _OUT)

    ids3, score, logit, quant, loss = pl.pallas_call(
        _vq_kernel,
        grid=(nb,),
        in_specs=[
            pl.BlockSpec((_TOK_BLK, _OUT), lambda i: (i, 0)),
            pl.BlockSpec((_DICT, _OUT), lambda i: (0, 0)),
            pl.BlockSpec((_VOCAB, _DICT), lambda i: (0, 0)),
        ],
        out_specs=[
            pl.BlockSpec((1, 1, _TOK_BLK), lambda i: (i, 0, 0)),
            pl.BlockSpec((_TOK_BLK, _VOCAB), lambda i: (i, 0)),
            pl.BlockSpec((_TOK_BLK, _VOCAB), lambda i: (i, 0)),
            pl.BlockSpec((_TOK_BLK, _DICT), lambda i: (i, 0)),
            pl.BlockSpec((1, 1), lambda i: (0, 0)),
        ],
        out_shape=[
            jax.ShapeDtypeStruct((nb, 1, _TOK_BLK), jnp.int32),
            jax.ShapeDtypeStruct((n_tok, _VOCAB), jnp.float32),
            jax.ShapeDtypeStruct((n_tok, _VOCAB), jnp.float32),
            jax.ShapeDtypeStruct((n_tok, _DICT), jnp.float32),
            jax.ShapeDtypeStruct((1, 1), jnp.float32),
        ],
        compiler_params=pltpu.CompilerParams(
            dimension_semantics=("arbitrary",),
        ),
    )(x2d, W_out, dictionary)

    ids = ids3.reshape(B, S)
    score = score.reshape(B, S, _VOCAB)
    logit = logit.reshape(B, S, _VOCAB)
    quantized = quant.reshape(B, S, _DICT)
    quantization_loss = loss[0, 0] * (1.25 / (n_tok * _DICT))
    return ids, score, logit, quantized, quantization_loss


# speedup vs baseline: 1.3427x; 1.3427x over previous
import jax
import jax.numpy as jnp
from jax.experimental import pallas as pl
from jax.experimental.pallas import tpu as pltpu

_VOCAB = 8192
_DICT = 64
_OUT = 384
_TOK_BLK = 128


def _vq_kernel(x_ref, w_ref, d_ref, ids_ref, score_ref, logit_ref,
               quant_ref, loss_ref):
    x = x_ref[...]
    w = w_ref[...]
    d = d_ref[...]
    cont = jax.lax.dot_general(
        x, w, (((1,), (1,)), ((), ())),
        preferred_element_type=jnp.float32)
    logit = jax.lax.dot_general(
        cont, d, (((1,), (1,)), ((), ())),
        preferred_element_type=jnp.float32)
    logit_ref[...] = logit

    m = jnp.max(logit, axis=1, keepdims=True)
    e = jnp.exp(logit - m)
    score_ref[...] = e / jnp.sum(e, axis=1, keepdims=True)

    iota = jax.lax.broadcasted_iota(jnp.int32, logit.shape, 1)
    ids = jnp.min(jnp.where(logit == m, iota, _VOCAB), axis=1)
    ids_ref[0, 0, :] = ids

    onehot = (iota == ids[:, None]).astype(jnp.float32)
    quant = jax.lax.dot_general(
        onehot, d, (((1,), (0,)), ((), ())),
        preferred_element_type=jnp.float32)
    quant_ref[...] = quant

    diff = cont - quant
    part = jnp.sum(diff * diff).reshape(1, 1)

    @pl.when(pl.program_id(0) == 0)
    def _():
        loss_ref[...] = jnp.zeros((1, 1), jnp.float32)

    loss_ref[...] += part


def kernel(x, W_out, dictionary):
    B, S, _ = x.shape
    n_tok = B * S
    nb = n_tok // _TOK_BLK
    x2d = x.reshape(n_tok, _OUT)

    ids3, score, logit, quant, loss = pl.pallas_call(
        _vq_kernel,
        grid=(nb,),
        in_specs=[
            pl.BlockSpec((_TOK_BLK, _OUT), lambda i: (i, 0)),
            pl.BlockSpec((_DICT, _OUT), lambda i: (0, 0)),
            pl.BlockSpec((_VOCAB, _DICT), lambda i: (0, 0)),
        ],
        out_specs=[
            pl.BlockSpec((1, 1, _TOK_BLK), lambda i: (i, 0, 0)),
            pl.BlockSpec((_TOK_BLK, _VOCAB), lambda i: (i, 0)),
            pl.BlockSpec((_TOK_BLK, _VOCAB), lambda i: (i, 0)),
            pl.BlockSpec((_TOK_BLK, _DICT), lambda i: (i, 0)),
            pl.BlockSpec((1, 1), lambda i: (0, 0)),
        ],
        out_shape=[
            jax.ShapeDtypeStruct((nb, 1, _TOK_BLK), jnp.int32),
            jax.ShapeDtypeStruct((n_tok, _VOCAB), jnp.float32),
            jax.ShapeDtypeStruct((n_tok, _VOCAB), jnp.float32),
            jax.ShapeDtypeStruct((n_tok, _DICT), jnp.float32),
            jax.ShapeDtypeStruct((1, 1), jnp.float32),
        ],
        compiler_params=pltpu.CompilerParams(
            dimension_semantics=("arbitrary",),
        ),
    )(x2d, W_out, dictionary)

    ids = ids3.reshape(B, S)
    score = score.reshape(B, S, _VOCAB)
    logit = logit.reshape(B, S, _VOCAB)
    quantized = quant.reshape(B, S, _DICT)
    quantization_loss = loss[0, 0] * (1.25 / (n_tok * _DICT))
    return ids, score, logit, quantized, quantization_loss


# speedup vs baseline: 1.4115x; 1.0512x over previous
import jax
import jax.numpy as jnp
from jax.experimental import pallas as pl
from jax.experimental.pallas import tpu as pltpu

_VOCAB = 8192
_DICT = 64
_OUT = 384
_TOK_BLK = 128


def _vq_kernel(x_ref, w_ref, d_ref, ids_ref, score_ref, logit_ref,
               quant_ref, loss_ref):
    x = x_ref[...]
    w = w_ref[...]
    d = d_ref[...]
    cont = jax.lax.dot_general(
        x, w, (((1,), (1,)), ((), ())),
        preferred_element_type=jnp.float32)
    logit = jax.lax.dot_general(
        cont, d, (((1,), (1,)), ((), ())),
        preferred_element_type=jnp.float32)
    logit_ref[...] = logit

    e = jnp.exp(logit)
    score_ref[...] = e * (1.0 / jnp.sum(e, axis=1, keepdims=True))

    ids = jnp.argmax(logit, axis=1).astype(jnp.int32)
    ids_ref[0, 0, :] = ids

    iota = jax.lax.broadcasted_iota(jnp.int32, logit.shape, 1)
    onehot = (iota == ids[:, None]).astype(jnp.float32)
    quant = jax.lax.dot_general(
        onehot, d, (((1,), (0,)), ((), ())),
        preferred_element_type=jnp.float32)
    quant_ref[...] = quant

    diff = cont - quant
    part = jnp.sum(diff * diff).reshape(1, 1)

    @pl.when(pl.program_id(0) == 0)
    def _():
        loss_ref[...] = jnp.zeros((1, 1), jnp.float32)

    loss_ref[...] += part


def kernel(x, W_out, dictionary):
    B, S, _ = x.shape
    n_tok = B * S
    nb = n_tok // _TOK_BLK
    x2d = x.reshape(n_tok, _OUT)

    ids3, score, logit, quant, loss = pl.pallas_call(
        _vq_kernel,
        grid=(nb,),
        in_specs=[
            pl.BlockSpec((_TOK_BLK, _OUT), lambda i: (i, 0)),
            pl.BlockSpec((_DICT, _OUT), lambda i: (0, 0)),
            pl.BlockSpec((_VOCAB, _DICT), lambda i: (0, 0)),
        ],
        out_specs=[
            pl.BlockSpec((1, 1, _TOK_BLK), lambda i: (i, 0, 0)),
            pl.BlockSpec((_TOK_BLK, _VOCAB), lambda i: (i, 0)),
            pl.BlockSpec((_TOK_BLK, _VOCAB), lambda i: (i, 0)),
            pl.BlockSpec((_TOK_BLK, _DICT), lambda i: (i, 0)),
            pl.BlockSpec((1, 1), lambda i: (0, 0)),
        ],
        out_shape=[
            jax.ShapeDtypeStruct((nb, 1, _TOK_BLK), jnp.int32),
            jax.ShapeDtypeStruct((n_tok, _VOCAB), jnp.float32),
            jax.ShapeDtypeStruct((n_tok, _VOCAB), jnp.float32),
            jax.ShapeDtypeStruct((n_tok, _DICT), jnp.float32),
            jax.ShapeDtypeStruct((1, 1), jnp.float32),
        ],
        compiler_params=pltpu.CompilerParams(
            dimension_semantics=("arbitrary",),
        ),
    )(x2d, W_out, dictionary)

    ids = ids3.reshape(B, S)
    score = score.reshape(B, S, _VOCAB)
    logit = logit.reshape(B, S, _VOCAB)
    quantized = quant.reshape(B, S, _DICT)
    quantization_loss = loss[0, 0] * (1.25 / (n_tok * _DICT))
    return ids, score, logit, quantized, quantization_loss


# speedup vs baseline: 1.5691x; 1.1116x over previous
import jax
import jax.numpy as jnp
from jax.experimental import pallas as pl
from jax.experimental.pallas import tpu as pltpu

_VOCAB = 8192
_DICT = 64
_OUT = 384
_TOK_BLK = 128


def _vq_kernel(x_ref, w_ref, d_ref, ids_ref, score_ref, logit_ref,
               quant_ref, loss_ref):
    x = x_ref[...]
    w = w_ref[...]
    d = d_ref[...]
    cont = jax.lax.dot_general(
        x, w, (((1,), (1,)), ((), ())),
        preferred_element_type=jnp.float32)
    logit = jax.lax.dot_general(
        cont, d, (((1,), (1,)), ((), ())),
        preferred_element_type=jnp.float32)
    logit_ref[...] = logit

    e = jnp.exp(logit)
    score_ref[...] = e * (1.0 / jnp.sum(e, axis=1, keepdims=True))

    ids = jnp.argmax(logit, axis=1).astype(jnp.int32)
    ids_ref[0, 0, :] = ids

    iota = jax.lax.broadcasted_iota(jnp.int32, logit.shape, 1)
    onehot = (iota == ids[:, None]).astype(jnp.float32)
    quant = jax.lax.dot_general(
        onehot, d, (((1,), (0,)), ((), ())),
        preferred_element_type=jnp.float32)
    quant_ref[...] = quant

    diff = cont - quant
    part = jnp.sum(diff * diff)
    loss_ref[0, 0, :] = jnp.full((128,), part, jnp.float32)


def kernel(x, W_out, dictionary):
    B, S, _ = x.shape
    n_tok = B * S
    nb = n_tok // _TOK_BLK
    x2d = x.reshape(n_tok, _OUT)

    ids3, score, logit, quant, loss = pl.pallas_call(
        _vq_kernel,
        grid=(nb,),
        in_specs=[
            pl.BlockSpec((_TOK_BLK, _OUT), lambda i: (i, 0)),
            pl.BlockSpec((_DICT, _OUT), lambda i: (0, 0)),
            pl.BlockSpec((_VOCAB, _DICT), lambda i: (0, 0)),
        ],
        out_specs=[
            pl.BlockSpec((1, 1, _TOK_BLK), lambda i: (i, 0, 0)),
            pl.BlockSpec((_TOK_BLK, _VOCAB), lambda i: (i, 0)),
            pl.BlockSpec((_TOK_BLK, _VOCAB), lambda i: (i, 0)),
            pl.BlockSpec((_TOK_BLK, _DICT), lambda i: (i, 0)),
            pl.BlockSpec((1, 1, 128), lambda i: (i, 0, 0)),
        ],
        out_shape=[
            jax.ShapeDtypeStruct((nb, 1, _TOK_BLK), jnp.int32),
            jax.ShapeDtypeStruct((n_tok, _VOCAB), jnp.float32),
            jax.ShapeDtypeStruct((n_tok, _VOCAB), jnp.float32),
            jax.ShapeDtypeStruct((n_tok, _DICT), jnp.float32),
            jax.ShapeDtypeStruct((nb, 1, 128), jnp.float32),
        ],
        compiler_params=pltpu.CompilerParams(
            dimension_semantics=("parallel",),
        ),
    )(x2d, W_out, dictionary)

    ids = ids3.reshape(B, S)
    score = score.reshape(B, S, _VOCAB)
    logit = logit.reshape(B, S, _VOCAB)
    quantized = quant.reshape(B, S, _DICT)
    quantization_loss = jnp.sum(loss[:, 0, 0]) * (1.25 / (n_tok * _DICT))
    return ids, score, logit, quantized, quantization_loss
